# baseline (speedup 1.0000x reference)
import functools

import jax
import jax.numpy as jnp
from jax import lax
from jax.experimental import pallas as pl
from jax.experimental.pallas import tpu as pltpu
from jax.experimental.pallas import tpu_sc as plsc

S = 2048
D = 1024
E = 64
FFW = 512
TOPK = 8
N_GROUP = 8
GROUP_SIZE = E // N_GROUP
TOPK_GROUP = 4
SCALE = 2.5

TS = 256
M = 256
P = S * TOPK + E * M
NT = P // M


def _router_body(x_ref, rw_ref, rb_ref, wsg_ref, wsu_ref, wsd_ref,
                 idx_ref, w_ref, shared_ref):
    xb = x_ref[...]
    logits = jnp.dot(xb, rw_ref[...], preferred_element_type=jnp.float32)
    scores = jax.nn.sigmoid(logits)
    swb = scores + rb_ref[...]

    iota8 = lax.broadcasted_iota(jnp.int32, (TS, N_GROUP), 1)
    neg = jnp.float32(-jnp.inf)

    gs_parts = []
    for g in range(N_GROUP):
        sg = swb[:, g * GROUP_SIZE:(g + 1) * GROUP_SIZE]
        m1 = jnp.max(sg, axis=-1, keepdims=True)
        i1 = jnp.min(jnp.where(sg == m1, iota8, N_GROUP), axis=-1, keepdims=True)
        m2 = jnp.max(jnp.where(iota8 == i1, neg, sg), axis=-1, keepdims=True)
        gs_parts.append(m1 + m2)
    gs = jnp.concatenate(gs_parts, axis=-1)

    sel = jnp.zeros((TS, N_GROUP), jnp.bool_)
    gw = gs
    for _ in range(TOPK_GROUP):
        gm = jnp.max(gw, axis=-1, keepdims=True)
        gi = jnp.min(jnp.where(gw == gm, iota8, N_GROUP), axis=-1, keepdims=True)
        sel = sel | (iota8 == gi)
        gw = jnp.where(iota8 == gi, neg, gw)

    ms = jnp.concatenate(
        [jnp.where(sel[:, g:g + 1], swb[:, g * GROUP_SIZE:(g + 1) * GROUP_SIZE], 0.0)
         for g in range(N_GROUP)], axis=-1)

    iota64 = lax.broadcasted_iota(jnp.int32, (TS, E), 1)
    idx_parts, w_parts = [], []
    mw = ms
    for _ in range(TOPK):
        m = jnp.max(mw, axis=-1, keepdims=True)
        ii = jnp.min(jnp.where(mw == m, iota64, E), axis=-1, keepdims=True)
        idx_parts.append(ii)
        w_parts.append(jnp.sum(jnp.where(iota64 == ii, scores, 0.0),
                               axis=-1, keepdims=True))
        mw = jnp.where(iota64 == ii, jnp.float32(-1.0), mw)

    idx = jnp.concatenate(idx_parts, axis=-1)
    w = jnp.concatenate(w_parts, axis=-1)
    w = SCALE * w / (jnp.sum(w, axis=-1, keepdims=True) + 1e-20)
    idx_ref[...] = idx
    w_ref[...] = w

    g_ = jnp.dot(xb, wsg_ref[...], preferred_element_type=jnp.float32)
    u_ = jnp.dot(xb, wsu_ref[...], preferred_element_type=jnp.float32)
    h_ = g_ * jax.nn.sigmoid(g_) * u_
    shared_ref[...] = jnp.dot(h_, wsd_ref[...], preferred_element_type=jnp.float32)


def _router_shared(xf, router_w, router_bias, ws_gate, ws_up, ws_down):
    grid = (S // TS,)
    return pl.pallas_call(
        _router_body,
        grid=grid,
        in_specs=[
            pl.BlockSpec((TS, D), lambda i: (i, 0)),
            pl.BlockSpec((D, E), lambda i: (0, 0)),
            pl.BlockSpec((1, E), lambda i: (0, 0)),
            pl.BlockSpec((D, FFW), lambda i: (0, 0)),
            pl.BlockSpec((D, FFW), lambda i: (0, 0)),
            pl.BlockSpec((FFW, D), lambda i: (0, 0)),
        ],
        out_specs=[
            pl.BlockSpec((TS, TOPK), lambda i: (i, 0)),
            pl.BlockSpec((TS, TOPK), lambda i: (i, 0)),
            pl.BlockSpec((TS, D), lambda i: (i, 0)),
        ],
        out_shape=[
            jax.ShapeDtypeStruct((S, TOPK), jnp.int32),
            jax.ShapeDtypeStruct((S, TOPK), jnp.float32),
            jax.ShapeDtypeStruct((S, D), jnp.float32),
        ],
    )(xf, router_w, router_bias.reshape(1, E), ws_gate, ws_up, ws_down)


def _gmm_body(te_ref, nu_ref, x_ref, wg_ref, wu_ref, wd_ref, out_ref):
    n = pl.program_id(0)

    @pl.when(n < nu_ref[0])
    def _():
        xb = x_ref[...].astype(jnp.bfloat16)
        wg = wg_ref[0].astype(jnp.bfloat16)
        wu = wu_ref[0].astype(jnp.bfloat16)
        wd = wd_ref[0].astype(jnp.bfloat16)
        g = jnp.dot(xb, wg, preferred_element_type=jnp.float32)
        u = jnp.dot(xb, wu, preferred_element_type=jnp.float32)
        h = (g * jax.nn.sigmoid(g) * u).astype(jnp.bfloat16)
        out_ref[...] = jnp.dot(h, wd, preferred_element_type=jnp.float32)


def _gmm(x_padded, we_gate, we_up, we_down, tile_expert, n_used):
    grid_spec = pltpu.PrefetchScalarGridSpec(
        num_scalar_prefetch=2,
        grid=(NT,),
        in_specs=[
            pl.BlockSpec((M, D), lambda n, te, nu: (n, 0)),
            pl.BlockSpec((1, D, FFW), lambda n, te, nu: (te[n], 0, 0)),
            pl.BlockSpec((1, D, FFW), lambda n, te, nu: (te[n], 0, 0)),
            pl.BlockSpec((1, FFW, D), lambda n, te, nu: (te[n], 0, 0)),
        ],
        out_specs=pl.BlockSpec((M, D), lambda n, te, nu: (n, 0)),
    )
    return pl.pallas_call(
        _gmm_body,
        grid_spec=grid_spec,
        out_shape=jax.ShapeDtypeStruct((P, D), jnp.float32),
    )(tile_expert, n_used, x_padded, we_gate, we_up, we_down)


def _dispatch(xf, row_src):
    return jnp.take(xf, row_src, axis=0)


def _combine(out_padded, shared, comb_idx, topk_w):
    rows = jnp.take(out_padded, comb_idx.reshape(-1), axis=0)
    rows = rows.reshape(S, TOPK, D) * topk_w[..., None]
    return shared + jnp.sum(rows, axis=1)


PB = 256
NSUB = S // PB


def _plan_body(idx_ref, dest_ref, te_ref, nu_ref, rank_ref):
    f32 = jnp.float32
    r256 = lax.broadcasted_iota(jnp.int32, (PB, PB), 0)
    c256 = lax.broadcasted_iota(jnp.int32, (PB, PB), 1)
    tril = (r256 > c256).astype(f32)
    lane64 = lax.broadcasted_iota(jnp.int32, (PB, E), 1)

    def onehot(k, sub):
        col = idx_ref[pl.ds(sub * PB, PB), k:k + 1]
        return (col == lane64).astype(f32)

    counts = jnp.zeros((1, E), f32)
    for k in range(TOPK):
        def body1(sub, base):
            oh = onehot(k, sub)
            cum = jnp.dot(tril, oh, preferred_element_type=f32) + base
            rank = jnp.sum(oh * cum, axis=1, keepdims=True)
            rank_ref[pl.ds(sub * PB, PB), k:k + 1] = rank.astype(jnp.int32)
            return base + jnp.sum(oh, axis=0, keepdims=True)
        counts = lax.fori_loop(0, NSUB, body1, counts)

    padded = jnp.floor((counts + (M - 1)) * (1.0 / M)) * M
    e_r = lax.broadcasted_iota(jnp.int32, (E, E), 0)
    e_c = lax.broadcasted_iota(jnp.int32, (E, E), 1)
    incl = (e_r <= e_c).astype(f32)
    cum_p = jnp.dot(padded, incl, preferred_element_type=f32)
    pstart = cum_p - padded

    for k in range(TOPK):
        def body2(sub, carry):
            oh = onehot(k, sub)
            base = jnp.sum(oh * pstart, axis=1, keepdims=True)
            rank = rank_ref[pl.ds(sub * PB, PB), k:k + 1].astype(f32)
            dest_ref[pl.ds(sub * PB, PB), k:k + 1] = (base + rank).astype(jnp.int32)
            return carry
        lax.fori_loop(0, NSUB, body2, 0)

    ident = (e_r == e_c).astype(f32)
    cum_p_col = lax.dot_general(ident, cum_p, (((1,), (1,)), ((), ())),
                                preferred_element_type=f32)
    ntile_lane = (lax.broadcasted_iota(jnp.int32, (E, NT), 1) * M).astype(f32)
    te = jnp.sum((ntile_lane >= cum_p_col).astype(f32), axis=0, keepdims=True)
    te_ref[...] = jnp.minimum(te, E - 1).astype(jnp.int32)
    nu = cum_p[:, E - 1:E] * (1.0 / M)
    nu_ref[...] = jnp.broadcast_to(nu, (1, 128)).astype(jnp.int32)


def _plan(topk_idx):
    dest, te, nu = pl.pallas_call(
        _plan_body,
        grid=(1,),
        in_specs=[pl.BlockSpec((S, TOPK), lambda i: (0, 0))],
        out_specs=[
            pl.BlockSpec((S, TOPK), lambda i: (0, 0)),
            pl.BlockSpec((1, NT), lambda i: (0, 0)),
            pl.BlockSpec((1, 128), lambda i: (0, 0)),
        ],
        out_shape=[
            jax.ShapeDtypeStruct((S, TOPK), jnp.int32),
            jax.ShapeDtypeStruct((1, NT), jnp.int32),
            jax.ShapeDtypeStruct((1, 128), jnp.int32),
        ],
        scratch_shapes=[pltpu.VMEM((S, TOPK), jnp.int32)],
    )(topk_idx)
    return dest, te.reshape(NT), nu.reshape(128)[:1]


def kernel(x, router_w, router_bias, we_gate, we_up, we_down,
           ws_gate, ws_up, ws_down):
    b, s, d = x.shape
    xf = x.reshape(s, d)

    topk_idx, topk_w, shared = _router_shared(
        xf, router_w, router_bias, ws_gate, ws_up, ws_down)

    dest, tile_expert, n_used = _plan(topk_idx)
    tok_ids = jnp.arange(S * TOPK, dtype=jnp.int32) // TOPK
    row_src = jnp.zeros((P,), jnp.int32).at[dest.reshape(-1)].set(tok_ids)

    x_padded = _dispatch(xf, row_src)
    out_padded = _gmm(x_padded, we_gate, we_up, we_down, tile_expert, n_used)
    out = _combine(out_padded, shared, dest.reshape(S, TOPK), topk_w)
    return out.reshape(b, s, d)
